# baseline (speedup 1.0000x reference)
import functools

import jax
import jax.numpy as jnp
from jax.experimental import pallas as pl
from jax.experimental.pallas import tpu as pltpu

E = 8
TOP_K = 2
ALPHA = 1.702
LIMIT = 7.0
LANES = 128


def _router_body(x_ref, rw_ref, rb_ref, w_ref):
    g = jnp.dot(x_ref[...], rw_ref[...].T, preferred_element_type=jnp.float32)
    g = g + rb_ref[...]
    t, l = g.shape
    lane = jax.lax.broadcasted_iota(jnp.int32, (t, l), 1)
    m1 = jnp.max(g, axis=1, keepdims=True)
    i1 = jnp.min(jnp.where(g == m1, lane, l), axis=1, keepdims=True)
    g2 = jnp.where(lane == i1, -jnp.inf, g)
    m2 = jnp.max(g2, axis=1, keepdims=True)
    i2 = jnp.min(jnp.where(g2 == m2, lane, l), axis=1, keepdims=True)
    w1 = 1.0 / (1.0 + jnp.exp(m2 - m1))
    w2 = 1.0 - w1
    w_ref[...] = jnp.where(lane == i1, w1, jnp.where(lane == i2, w2, 0.0))


def _moe_body(x_ref, wg_ref, wu_ref, bg_ref, bu_ref, w2_ref, b2_ref, wts_ref,
              out_ref, *, tb):
    e = pl.program_id(0)
    t = pl.program_id(1)

    @pl.when((e == 0) & (t == 0))
    def _init():
        out_ref[...] = jnp.zeros_like(out_ref)

    x = x_ref[...]
    gate = jnp.dot(x, wg_ref[0].T, preferred_element_type=jnp.float32) + bg_ref[0]
    up = jnp.dot(x, wu_ref[0].T, preferred_element_type=jnp.float32) + bu_ref[0]
    gate = jnp.minimum(gate, LIMIT)
    up = jnp.clip(up, -LIMIT, LIMIT)
    glu = gate * jax.nn.sigmoid(ALPHA * gate)
    act = (up + 1.0) * glu
    y = jnp.dot(act.astype(w2_ref.dtype), w2_ref[0].T,
                preferred_element_type=jnp.float32) + b2_ref[0]
    lane = jax.lax.broadcasted_iota(jnp.int32, wts_ref.shape, 1)
    wcol = jnp.sum(jnp.where(lane == e, wts_ref[...], 0.0), axis=1, keepdims=True)
    out_ref[pl.ds(t * tb, tb), :] += wcol * y


def kernel(x, router_w, router_b, w13, b13, w2, b2):
    T, H = x.shape
    I2 = w13.shape[1]
    I = I2 // 2

    rwp = jnp.zeros((LANES, H), jnp.float32).at[:E].set(router_w)
    rbp = jnp.full((1, LANES), -1e30, jnp.float32).at[0, :E].set(router_b)
    weights = pl.pallas_call(
        _router_body,
        out_shape=jax.ShapeDtypeStruct((T, LANES), jnp.float32),
    )(x, rwp, rbp)

    wg = w13[:, 0::2, :].astype(jnp.bfloat16)
    wu = w13[:, 1::2, :].astype(jnp.bfloat16)
    bg = b13[:, 0::2].reshape(E, 1, I)
    bu = b13[:, 1::2].reshape(E, 1, I)
    b2r = b2.reshape(E, 1, H)
    w2c = w2.astype(jnp.bfloat16)
    xc = x.astype(jnp.bfloat16)

    TB = 256
    grid = (E, T // TB)
    out = pl.pallas_call(
        functools.partial(_moe_body, tb=TB),
        grid=grid,
        in_specs=[
            pl.BlockSpec((TB, H), lambda e, t: (t, 0)),
            pl.BlockSpec((1, I, H), lambda e, t: (e, 0, 0)),
            pl.BlockSpec((1, I, H), lambda e, t: (e, 0, 0)),
            pl.BlockSpec((1, 1, I), lambda e, t: (e, 0, 0)),
            pl.BlockSpec((1, 1, I), lambda e, t: (e, 0, 0)),
            pl.BlockSpec((1, H, I), lambda e, t: (e, 0, 0)),
            pl.BlockSpec((1, 1, H), lambda e, t: (e, 0, 0)),
            pl.BlockSpec((TB, LANES), lambda e, t: (t, 0)),
        ],
        out_specs=pl.BlockSpec((T, H), lambda e, t: (0, 0)),
        out_shape=jax.ShapeDtypeStruct((T, H), jnp.float32),
    )(xc, wg, wu, bg, bu, w2c, b2r, weights)
    return out


# speedup vs baseline: 1.5412x; 1.5412x over previous
import functools

import jax
import jax.numpy as jnp
from jax.experimental import pallas as pl
from jax.experimental.pallas import tpu as pltpu

E = 8
TOP_K = 2
ALPHA = 1.702
LIMIT = 7.0
LANES = 128


def _router_body(x_ref, rw_ref, rb_ref, w_ref):
    g = jnp.dot(x_ref[...], rw_ref[...].T, preferred_element_type=jnp.float32)
    g = g + rb_ref[...]
    t, l = g.shape
    lane = jax.lax.broadcasted_iota(jnp.int32, (t, l), 1)
    m1 = jnp.max(g, axis=1, keepdims=True)
    i1 = jnp.min(jnp.where(g == m1, lane, l), axis=1, keepdims=True)
    g2 = jnp.where(lane == i1, -jnp.inf, g)
    m2 = jnp.max(g2, axis=1, keepdims=True)
    i2 = jnp.min(jnp.where(g2 == m2, lane, l), axis=1, keepdims=True)
    w1 = 1.0 / (1.0 + jnp.exp(m2 - m1))
    w2 = 1.0 - w1
    w_ref[...] = jnp.where(lane == i1, w1, jnp.where(lane == i2, w2, 0.0))


def _moe_body(x_ref, w13_ref, bg_ref, bu_ref, w2_ref, b2_ref, wts_ref,
              out_ref, xb_ref, wgb_ref, wub_ref, w2b_ref, *, tb):
    e = pl.program_id(0)
    t = pl.program_id(1)

    @pl.when((e == 0) & (t == 0))
    def _init():
        out_ref[...] = jnp.zeros_like(out_ref)
        xb_ref[...] = x_ref[...].astype(jnp.bfloat16)

    @pl.when(t == 0)
    def _cast_weights():
        wgb_ref[...] = w13_ref[0, :, 0, :].astype(jnp.bfloat16)
        wub_ref[...] = w13_ref[0, :, 1, :].astype(jnp.bfloat16)
        w2b_ref[...] = w2_ref[0].astype(jnp.bfloat16)

    x = xb_ref[pl.ds(t * tb, tb), :]
    gate = jnp.dot(x, wgb_ref[...].T, preferred_element_type=jnp.float32)
    gate = gate + bg_ref[0]
    up = jnp.dot(x, wub_ref[...].T, preferred_element_type=jnp.float32)
    up = up + bu_ref[0]
    gate = jnp.minimum(gate, LIMIT)
    up = jnp.clip(up, -LIMIT, LIMIT)
    glu = gate * jax.nn.sigmoid(ALPHA * gate)
    act = (up + 1.0) * glu
    y = jnp.dot(act.astype(jnp.bfloat16), w2b_ref[...].T,
                preferred_element_type=jnp.float32) + b2_ref[0]
    lane = jax.lax.broadcasted_iota(jnp.int32, wts_ref.shape, 1)
    wcol = jnp.sum(jnp.where(lane == e, wts_ref[...], 0.0), axis=1, keepdims=True)
    out_ref[pl.ds(t * tb, tb), :] += wcol * y


def kernel(x, router_w, router_b, w13, b13, w2, b2):
    T, H = x.shape
    I2 = w13.shape[1]
    I = I2 // 2

    rwp = jnp.zeros((LANES, H), jnp.float32).at[:E].set(router_w)
    rbp = jnp.full((1, LANES), -1e30, jnp.float32).at[0, :E].set(router_b)
    weights = pl.pallas_call(
        _router_body,
        out_shape=jax.ShapeDtypeStruct((T, LANES), jnp.float32),
    )(x, rwp, rbp)

    w13r = w13.reshape(E, I, 2, H)
    bg = b13[:, 0::2].reshape(E, 1, I)
    bu = b13[:, 1::2].reshape(E, 1, I)
    b2r = b2.reshape(E, 1, H)

    TB = 256
    grid = (E, T // TB)
    out = pl.pallas_call(
        functools.partial(_moe_body, tb=TB),
        grid=grid,
        in_specs=[
            pl.BlockSpec((T, H), lambda e, t: (0, 0)),
            pl.BlockSpec((1, I, 2, H), lambda e, t: (e, 0, 0, 0)),
            pl.BlockSpec((1, 1, I), lambda e, t: (e, 0, 0)),
            pl.BlockSpec((1, 1, I), lambda e, t: (e, 0, 0)),
            pl.BlockSpec((1, H, I), lambda e, t: (e, 0, 0)),
            pl.BlockSpec((1, 1, H), lambda e, t: (e, 0, 0)),
            pl.BlockSpec((TB, LANES), lambda e, t: (t, 0)),
        ],
        out_specs=pl.BlockSpec((T, H), lambda e, t: (0, 0)),
        out_shape=jax.ShapeDtypeStruct((T, H), jnp.float32),
        scratch_shapes=[
            pltpu.VMEM((T, H), jnp.bfloat16),
            pltpu.VMEM((I, H), jnp.bfloat16),
            pltpu.VMEM((I, H), jnp.bfloat16),
            pltpu.VMEM((H, I), jnp.bfloat16),
        ],
    )(x, w13r, bg, bu, w2, b2r, weights)
    return out


# speedup vs baseline: 1.8876x; 1.2248x over previous
import functools

import jax
import jax.numpy as jnp
from jax.experimental import pallas as pl
from jax.experimental.pallas import tpu as pltpu

E = 8
TOP_K = 2
ALPHA = 1.702
LIMIT = 7.0
LANES = 128


def _router_body(x_ref, rw_ref, rb_ref, w_ref):
    g = jnp.dot(x_ref[...], rw_ref[...].T, preferred_element_type=jnp.float32)
    g = g + rb_ref[...]
    t, l = g.shape
    lane = jax.lax.broadcasted_iota(jnp.int32, (t, l), 1)
    m1 = jnp.max(g, axis=1, keepdims=True)
    i1 = jnp.min(jnp.where(g == m1, lane, l), axis=1, keepdims=True)
    g2 = jnp.where(lane == i1, -jnp.inf, g)
    m2 = jnp.max(g2, axis=1, keepdims=True)
    i2 = jnp.min(jnp.where(g2 == m2, lane, l), axis=1, keepdims=True)
    w1 = 1.0 / (1.0 + jnp.exp(m2 - m1))
    w2 = 1.0 - w1
    w_ref[...] = jnp.where(lane == i1, w1, jnp.where(lane == i2, w2, 0.0))


def _moe_body(x_ref, w13_ref, bg_ref, bu_ref, w2_ref, b2_ref, wts_ref,
              out_ref, xb_ref, wgb_ref, wub_ref, w2b_ref, *, tb):
    e = pl.program_id(0)
    t = pl.program_id(1)

    @pl.when((e == 0) & (t == 0))
    def _init():
        out_ref[...] = jnp.zeros_like(out_ref)
        xb_ref[...] = x_ref[...].astype(jnp.bfloat16)

    h = w2_ref.shape[1]

    @pl.when(t == 0)
    def _cast_weights():
        wgb_ref[...] = w13_ref[0, :, :h].astype(jnp.bfloat16)
        wub_ref[...] = w13_ref[0, :, h:].astype(jnp.bfloat16)
        w2b_ref[...] = w2_ref[0].astype(jnp.bfloat16)

    x = xb_ref[pl.ds(t * tb, tb), :]
    gate = jnp.dot(x, wgb_ref[...].T, preferred_element_type=jnp.float32)
    gate = gate + bg_ref[0]
    up = jnp.dot(x, wub_ref[...].T, preferred_element_type=jnp.float32)
    up = up + bu_ref[0]
    gate = jnp.minimum(gate, LIMIT)
    up = jnp.clip(up, -LIMIT, LIMIT)
    glu = gate * jax.nn.sigmoid(ALPHA * gate)
    act = (up + 1.0) * glu
    y = jnp.dot(act.astype(jnp.bfloat16), w2b_ref[...].T,
                preferred_element_type=jnp.float32) + b2_ref[0]
    lane = jax.lax.broadcasted_iota(jnp.int32, wts_ref.shape, 1)
    wcol = jnp.sum(jnp.where(lane == e, wts_ref[...], 0.0), axis=1, keepdims=True)
    out_ref[pl.ds(t * tb, tb), :] += wcol * y


def kernel(x, router_w, router_b, w13, b13, w2, b2):
    T, H = x.shape
    I2 = w13.shape[1]
    I = I2 // 2

    rwp = jnp.zeros((LANES, H), jnp.float32).at[:E].set(router_w)
    rbp = jnp.full((1, LANES), -1e30, jnp.float32).at[0, :E].set(router_b)
    weights = pl.pallas_call(
        _router_body,
        out_shape=jax.ShapeDtypeStruct((T, LANES), jnp.float32),
    )(x, rwp, rbp)

    w13r = w13.reshape(E, I, 2 * H)
    bg = b13[:, 0::2].reshape(E, 1, I)
    bu = b13[:, 1::2].reshape(E, 1, I)
    b2r = b2.reshape(E, 1, H)

    TB = 256
    grid = (E, T // TB)
    out = pl.pallas_call(
        functools.partial(_moe_body, tb=TB),
        grid=grid,
        in_specs=[
            pl.BlockSpec((T, H), lambda e, t: (0, 0)),
            pl.BlockSpec((1, I, 2 * H), lambda e, t: (e, 0, 0)),
            pl.BlockSpec((1, 1, I), lambda e, t: (e, 0, 0)),
            pl.BlockSpec((1, 1, I), lambda e, t: (e, 0, 0)),
            pl.BlockSpec((1, H, I), lambda e, t: (e, 0, 0)),
            pl.BlockSpec((1, 1, H), lambda e, t: (e, 0, 0)),
            pl.BlockSpec((TB, LANES), lambda e, t: (t, 0)),
        ],
        out_specs=pl.BlockSpec((T, H), lambda e, t: (0, 0)),
        out_shape=jax.ShapeDtypeStruct((T, H), jnp.float32),
        scratch_shapes=[
            pltpu.VMEM((T, H), jnp.bfloat16),
            pltpu.VMEM((I, H), jnp.bfloat16),
            pltpu.VMEM((I, H), jnp.bfloat16),
            pltpu.VMEM((H, I), jnp.bfloat16),
        ],
    )(x, w13r, bg, bu, w2, b2r, weights)
    return out
